# baseline (speedup 1.0000x reference)
import functools

import jax
import jax.numpy as jnp
from jax.experimental import pallas as pl
from jax.experimental.pallas import tpu as pltpu

HIDDEN = 4096
NUM_EXPERTS = 64
BLOCK_TOKENS = 1024
HALF = HIDDEN // 2


def _router_kernel(xa_ref, xb_ref, w_ref, logits_ref, probs_ref):
    w = w_ref[...].astype(jnp.bfloat16)
    la = jax.lax.dot_general(
        xa_ref[...].astype(jnp.bfloat16),
        w[:, :HALF],
        (((1,), (1,)), ((), ())),
        preferred_element_type=jnp.float32,
    )
    lb = jax.lax.dot_general(
        xb_ref[...].astype(jnp.bfloat16),
        w[:, HALF:],
        (((1,), (1,)), ((), ())),
        preferred_element_type=jnp.float32,
    )
    logits = la + lb
    m = jnp.max(logits, axis=-1, keepdims=True)
    e = jnp.exp(logits - m)
    probs = e / jnp.sum(e, axis=-1, keepdims=True)
    logits_ref[...] = logits
    probs_ref[...] = probs


@functools.partial(jax.jit, static_argnames=())
def kernel(hidden_states, W):
    tokens, hidden = hidden_states.shape
    num_experts = W.shape[0]
    grid = (tokens // BLOCK_TOKENS,)
    out_shape = jax.ShapeDtypeStruct((tokens, num_experts), jnp.float32)
    logits, probs = pl.pallas_call(
        _router_kernel,
        grid=grid,
        in_specs=[
            pl.BlockSpec((BLOCK_TOKENS, HALF), lambda i: (i, 0)),
            pl.BlockSpec((BLOCK_TOKENS, HALF), lambda i: (i, 1)),
            pl.BlockSpec((num_experts, hidden), lambda i: (0, 0)),
        ],
        out_specs=[
            pl.BlockSpec((BLOCK_TOKENS, num_experts), lambda i: (i, 0)),
            pl.BlockSpec((BLOCK_TOKENS, num_experts), lambda i: (i, 0)),
        ],
        out_shape=[out_shape, out_shape],
        compiler_params=pltpu.CompilerParams(
            dimension_semantics=("parallel",),
        ),
    )(hidden_states, hidden_states, W)
    return (logits, probs)
